# baseline (speedup 1.0000x reference)
import functools

import jax
import jax.numpy as jnp
from jax import lax
from jax.experimental import pallas as pl
from jax.experimental.pallas import tpu as pltpu
from jax.experimental.pallas import tpu_sc as plsc

B = 4096
T = 200
D = 64

NC = 2
NS = 16
NW = NC * NS
BPW = B // NW

IDX_ROWS = 2
IDX_COLS = T // IDX_ROWS


def _sc_body(rid_hbm, table_hbm, pos_hbm, out_hbm, idx_v, rows_v, outb_v,
             pos_v, post_v, sem):
    wid = lax.axis_index("s") * NC + lax.axis_index("c")
    base = wid * BPW
    pltpu.sync_copy(pos_hbm, pos_v)
    iota = lax.iota(jnp.int32, 16)
    iota_t = iota * T
    iota_d = iota * D

    t_offs = tuple(range(0, T - 16, 16)) + (T - 16,)

    def pos_body(d, carry):
        for t0 in t_offs:
            vals = pos_v[pl.ds(d * T + t0, 16)]
            plsc.store_scatter(post_v, [iota_d + (t0 * D + d)], vals)
        return carry

    lax.fori_loop(0, D, pos_body, 0)

    def batch_body(i, carry):
        b = base + i
        pltpu.sync_copy(rid_hbm.at[b], idx_v)
        cp0 = pltpu.async_copy(table_hbm.at[idx_v.at[0]],
                               rows_v.at[pl.ds(0, IDX_COLS)], sem)
        cp1 = pltpu.async_copy(table_hbm.at[idx_v.at[1]],
                               rows_v.at[pl.ds(IDX_COLS, IDX_COLS)], sem)
        cp0.wait()
        cp1.wait()

        def t_body(t, tcarry):
            for d0 in range(0, D, 16):
                vals = rows_v[t, pl.ds(d0, 16)] + post_v[pl.ds(t * D + d0, 16)]
                plsc.store_scatter(outb_v, [iota_t + (d0 * T + t)], vals)
            return tcarry

        lax.fori_loop(0, T, t_body, 0)
        pltpu.sync_copy(outb_v, out_hbm.at[b])
        return carry

    lax.fori_loop(0, BPW, batch_body, 0)


def kernel(rid, table, pos):
    rid3 = rid.astype(jnp.int32).reshape(B, IDX_ROWS, IDX_COLS)
    pos_flat = pos.reshape(D * T)
    mesh = plsc.VectorSubcoreMesh(core_axis_name="c", subcore_axis_name="s",
                                  num_cores=NC, num_subcores=NS)
    k = functools.partial(
        pl.kernel,
        out_type=jax.ShapeDtypeStruct((B, D * T), jnp.float32),
        mesh=mesh,
        compiler_params=pltpu.CompilerParams(needs_layout_passes=False,
                                             use_tc_tiling_on_sc=False),
        scratch_types=[
            pltpu.VMEM((IDX_ROWS, IDX_COLS), jnp.int32),
            pltpu.VMEM((T, D), jnp.float32),
            pltpu.VMEM((D * T,), jnp.float32),
            pltpu.VMEM((D * T,), jnp.float32),
            pltpu.VMEM((T * D,), jnp.float32),
            pltpu.SemaphoreType.DMA,
        ],
    )(_sc_body)
    return k(rid3, table, pos_flat).reshape(B, D, T)


# speedup vs baseline: 1.1944x; 1.1944x over previous
import functools

import jax
import jax.numpy as jnp
from jax import lax
from jax.experimental import pallas as pl
from jax.experimental.pallas import tpu as pltpu
from jax.experimental.pallas import tpu_sc as plsc

B = 4096
T = 200
D = 64

NC = 2
NS = 16
NW = NC * NS
BPW = B // NW

IDX_ROWS = 2
IDX_COLS = T // IDX_ROWS


def _sc_body(rid_hbm, table_hbm, pos_hbm, out_hbm, idx_v, rows_v, outb_v,
             pos_v, post_v, gsems, osems):
    wid = lax.axis_index("s") * NC + lax.axis_index("c")
    base = wid * BPW
    iota = lax.iota(jnp.int32, 16)
    iota_t = iota * T
    iota_d = iota * D

    t_offs = tuple(range(0, T - 16, 16)) + (T - 16,)

    pltpu.sync_copy(rid_hbm.at[pl.ds(base, BPW)], idx_v)

    def start_gather(i, p):
        for c in range(IDX_ROWS):
            pltpu.async_copy(
                table_hbm.at[idx_v.at[i].at[c]],
                rows_v.at[p].at[pl.ds(c * IDX_COLS, IDX_COLS)],
                gsems.at[p])

    def wait_gather(i, p):
        for c in range(IDX_ROWS):
            pltpu.make_async_copy(
                table_hbm.at[idx_v.at[i].at[c]],
                rows_v.at[p].at[pl.ds(c * IDX_COLS, IDX_COLS)],
                gsems.at[p]).wait()

    def wait_store(b, p):
        pltpu.make_async_copy(outb_v.at[p], out_hbm.at[b], osems.at[p]).wait()

    start_gather(0, 0)
    start_gather(1, 1)
    pltpu.sync_copy(pos_hbm, pos_v)

    def pos_body(d, carry):
        for t0 in t_offs:
            vals = pos_v[pl.ds(d * T + t0, 16)]
            plsc.store_scatter(post_v, [iota_d + (t0 * D + d)], vals)
        return carry

    lax.fori_loop(0, D, pos_body, 0)

    def pair_body(j, carry):
        for p in range(2):
            i = 2 * j + p
            wait_gather(i, p)

            @pl.when(j > 0)
            def _():
                wait_store(base + i - 2, p)

            def t_body(t, tcarry):
                for d0 in range(0, D, 16):
                    vals = (rows_v[p, t, pl.ds(d0, 16)]
                            + post_v[pl.ds(t * D + d0, 16)])
                    plsc.store_scatter(outb_v.at[p],
                                       [iota_t + (d0 * T + t)], vals)
                return tcarry

            lax.fori_loop(0, T, t_body, 0)

            @pl.when(j < BPW // 2 - 1)
            def _():
                start_gather(i + 2, p)

            pltpu.async_copy(outb_v.at[p], out_hbm.at[base + i], osems.at[p])
        return carry

    lax.fori_loop(0, BPW // 2, pair_body, 0)
    wait_store(base + BPW - 2, 0)
    wait_store(base + BPW - 1, 1)


def kernel(rid, table, pos):
    rid3 = rid.astype(jnp.int32).reshape(B, IDX_ROWS, IDX_COLS)
    pos_flat = pos.reshape(D * T)
    mesh = plsc.VectorSubcoreMesh(core_axis_name="c", subcore_axis_name="s",
                                  num_cores=NC, num_subcores=NS)
    k = functools.partial(
        pl.kernel,
        out_type=jax.ShapeDtypeStruct((B, D * T), jnp.float32),
        mesh=mesh,
        compiler_params=pltpu.CompilerParams(needs_layout_passes=False,
                                             use_tc_tiling_on_sc=False),
        scratch_types=[
            pltpu.VMEM((BPW, IDX_ROWS, IDX_COLS), jnp.int32),
            pltpu.VMEM((2, T, D), jnp.float32),
            pltpu.VMEM((2, D * T), jnp.float32),
            pltpu.VMEM((D * T,), jnp.float32),
            pltpu.VMEM((T * D,), jnp.float32),
            pltpu.SemaphoreType.DMA((2,)),
            pltpu.SemaphoreType.DMA((2,)),
        ],
    )(_sc_body)
    return k(rid3, table, pos_flat).reshape(B, D, T)
